# Initial kernel scaffold; baseline (speedup 1.0000x reference)
import jax
import jax.numpy as jnp
from jax import lax
from jax.experimental import pallas as pl
from jax.experimental.pallas import tpu as pltpu
from jax.experimental.pallas import tpu_sc as plsc

N = 10000
D = 128
E = 320000
EPS = 1e-5

NC = 2
NS = 16
L = 16
NW = NC * NS
EW = E // NW
K = 80
CH = EW // K
RPT = N // NS
ZB = 125
ZC = RPT // ZB

BR = 1000
NB = N // BR


def _seg_sum_call(count: bool):
    mesh = plsc.VectorSubcoreMesh(
        core_axis_name="c", subcore_axis_name="s", num_cores=NC, num_subcores=NS
    )
    out_type = [jax.ShapeDtypeStruct((NC, N, D), jnp.float32)]
    if count:
        out_type.append(jax.ShapeDtypeStruct((NC, N, L), jnp.float32))
    scratch = [
        pltpu.VMEM((CH, K), jnp.int32),
        pltpu.VMEM((CH, K), jnp.int32),
        pltpu.VMEM((K, D), jnp.float32),
        pltpu.VMEM((K, D), jnp.float32),
        pltpu.VMEM((ZB, D), jnp.float32),
        pltpu.VMEM_SHARED((N, D), jnp.float32),
        pltpu.SemaphoreType.DMA,
        pltpu.SemaphoreType.DMA,
    ]
    if count:
        scratch += [
            pltpu.VMEM((K, L), jnp.float32),
            pltpu.VMEM((ZB, L), jnp.float32),
            pltpu.VMEM_SHARED((N, L), jnp.float32),
        ]

    def body(h_hbm, srcw_hbm, dstw_hbm, p_hbm, *rest):
        if count:
            (c_hbm, src_v, dst_v, rows0, rows1, zeros_v, acc_sh, sem0, sem1,
             ones_v, zcnt_v, cnt_sh) = rest
        else:
            (src_v, dst_v, rows0, rows1, zeros_v, acc_sh, sem0, sem1) = rest
        c = lax.axis_index("c")
        s = lax.axis_index("s")
        wid = c * NS + s

        zvec = jnp.zeros((L,), jnp.float32)

        def zrow(i, _):
            for j in range(D // L):
                zeros_v[i, pl.ds(j * L, L)] = zvec
            return 0

        lax.fori_loop(0, ZB, zrow, 0)
        if count:
            def orow(i, _):
                @pl.when(i < K)
                def _():
                    ones_v[i, :] = jnp.full((L,), 1.0, jnp.float32)

                zcnt_v[i, :] = zvec
                return 0

            lax.fori_loop(0, max(K, ZB), orow, 0)

        for k in range(ZC):
            pltpu.sync_copy(zeros_v, acc_sh.at[pl.ds(s * RPT + k * ZB, ZB)])
        if count:
            for k in range(ZC):
                pltpu.sync_copy(zcnt_v, cnt_sh.at[pl.ds(s * RPT + k * ZB, ZB)])
        plsc.subcore_barrier()

        pltpu.sync_copy(srcw_hbm.at[wid], src_v)
        pltpu.sync_copy(dstw_hbm.at[wid], dst_v)

        def scat(rows, j):
            pltpu.sync_copy(rows, acc_sh.at[dst_v.at[j]], add=True)
            if count:
                pltpu.sync_copy(ones_v, cnt_sh.at[dst_v.at[j]], add=True)

        pltpu.async_copy(h_hbm.at[src_v.at[0]], rows0, sem0)

        def pair(i, _):
            j0 = 2 * i
            pltpu.async_copy(h_hbm.at[src_v.at[j0 + 1]], rows1, sem1)
            pltpu.make_async_copy(h_hbm.at[src_v.at[j0]], rows0, sem0).wait()
            scat(rows0, j0)

            @pl.when(j0 + 2 < CH)
            def _():
                pltpu.async_copy(h_hbm.at[src_v.at[j0 + 2]], rows0, sem0)

            pltpu.make_async_copy(h_hbm.at[src_v.at[j0 + 1]], rows1, sem1).wait()
            scat(rows1, j0 + 1)
            return 0

        lax.fori_loop(0, CH // 2, pair, 0)
        if CH % 2:
            pltpu.make_async_copy(h_hbm.at[src_v.at[CH - 1]], rows0, sem0).wait()
            scat(rows0, CH - 1)

        plsc.subcore_barrier()

        for k in range(ZC):
            sl = pl.ds(s * RPT + k * ZB, ZB)
            pltpu.sync_copy(acc_sh.at[sl], p_hbm.at[c].at[sl])
        if count:
            pltpu.sync_copy(cnt_sh.at[pl.ds(s * RPT, RPT)],
                            c_hbm.at[c].at[pl.ds(s * RPT, RPT)])

    return pl.kernel(body, out_type=tuple(out_type), mesh=mesh,
                     scratch_types=scratch)


def _mm_stats_call(h, p, cnts, wl, wr, b):
    def body(p_ref, c_ref, h_ref, wl_ref, wr_ref, b_ref, t_ref, st_ref):
        i = pl.program_id(0)
        cnt = jnp.maximum(c_ref[0, :, 0:1] + c_ref[1, :, 0:1], 1.0)
        agg = (p_ref[0] + p_ref[1]) / cnt
        t = (jnp.dot(agg, wl_ref[...], preferred_element_type=jnp.float32)
             + jnp.dot(h_ref[...], wr_ref[...], preferred_element_type=jnp.float32)
             + b_ref[...])
        t_ref[...] = t

        @pl.when(i == 0)
        def _():
            st_ref[...] = jnp.zeros_like(st_ref)

        st_ref[0:1, :] += jnp.sum(t, axis=0, keepdims=True)
        st_ref[1:2, :] += jnp.sum(t * t, axis=0, keepdims=True)

    return pl.pallas_call(
        body,
        grid=(NB,),
        in_specs=[
            pl.BlockSpec((NC, BR, D), lambda i: (0, i, 0)),
            pl.BlockSpec((NC, BR, L), lambda i: (0, i, 0)),
            pl.BlockSpec((BR, D), lambda i: (i, 0)),
            pl.BlockSpec((D, D), lambda i: (0, 0)),
            pl.BlockSpec((D, D), lambda i: (0, 0)),
            pl.BlockSpec((1, D), lambda i: (0, 0)),
        ],
        out_specs=[
            pl.BlockSpec((BR, D), lambda i: (i, 0)),
            pl.BlockSpec((8, D), lambda i: (0, 0)),
        ],
        out_shape=[
            jax.ShapeDtypeStruct((N, D), jnp.float32),
            jax.ShapeDtypeStruct((8, D), jnp.float32),
        ],
    )(p, cnts, h, wl, wr, b)


def _norm_call(t, st, g, bt, relu: bool):
    def body(t_ref, st_ref, g_ref, bt_ref, o_ref):
        m = st_ref[0:1, :] * (1.0 / N)
        v = st_ref[1:2, :] * (1.0 / N) - m * m
        inv = lax.rsqrt(v + EPS)
        y = (t_ref[...] - m) * (inv * g_ref[...]) + bt_ref[...]
        if relu:
            y = jnp.maximum(y, 0.0)
        o_ref[...] = y

    return pl.pallas_call(
        body,
        grid=(NB,),
        in_specs=[
            pl.BlockSpec((BR, D), lambda i: (i, 0)),
            pl.BlockSpec((8, D), lambda i: (0, 0)),
            pl.BlockSpec((1, D), lambda i: (0, 0)),
            pl.BlockSpec((1, D), lambda i: (0, 0)),
        ],
        out_specs=pl.BlockSpec((BR, D), lambda i: (i, 0)),
        out_shape=jax.ShapeDtypeStruct((N, D), jnp.float32),
    )(t, st, g, bt)


@jax.jit
def kernel(x, edge_index, W1l, W1r, b1, g1, bt1, W2l, W2r, b2, g2, bt2):
    src = edge_index[0].astype(jnp.int32).reshape(NW, CH, K)
    dst = edge_index[1].astype(jnp.int32).reshape(NW, CH, K)
    b1r = b1.reshape(1, D)
    g1r = g1.reshape(1, D)
    bt1r = bt1.reshape(1, D)
    b2r = b2.reshape(1, D)
    g2r = g2.reshape(1, D)
    bt2r = bt2.reshape(1, D)

    p1, cnts = _seg_sum_call(count=True)(x, src, dst)
    t1, st1 = _mm_stats_call(x, p1, cnts, W1l, W1r, b1r)
    h1 = _norm_call(t1, st1, g1r, bt1r, relu=True)
    p2 = _seg_sum_call(count=False)(h1, src, dst)
    t2, st2 = _mm_stats_call(h1, p2, cnts, W2l, W2r, b2r)
    return _norm_call(t2, st2, g2r, bt2r, relu=False)


# speedup vs baseline: 8.9334x; 8.9334x over previous
import jax
import jax.numpy as jnp
from jax import lax
from jax.experimental import pallas as pl
from jax.experimental.pallas import tpu as pltpu
from jax.experimental.pallas import tpu_sc as plsc

N = 10000
D = 128
E = 320000
EPS = 1e-5

NC = 2
NS = 16
L = 16
NW = NC * NS
EW = E // NW
K = 80
CH = EW // K
NP = 10240
RPT = NP // NS

BR = 1000
NB = N // BR


def _seg_sum_call():
    mesh = plsc.VectorSubcoreMesh(
        core_axis_name="c", subcore_axis_name="s", num_cores=NC, num_subcores=NS
    )
    out_type = jax.ShapeDtypeStruct((NC, NP, D), jnp.float32)
    scratch = [
        pltpu.VMEM((EW,), jnp.int32),
        pltpu.VMEM((CH, K), jnp.int32),
        pltpu.VMEM((K, D), jnp.float32),
        pltpu.VMEM((K, D), jnp.float32),
        pltpu.VMEM_SHARED((NP, D), jnp.float32),
        pltpu.SemaphoreType.DMA,
        pltpu.SemaphoreType.DMA,
    ]

    def body(h_hbm, srcw_hbm, dstw_hbm, zacc_hbm, p_hbm,
             src_v, dst_v, rows0, rows1, acc_sh, sem0, sem1):
        c = lax.axis_index("c")
        s = lax.axis_index("s")
        wid = c * NS + s
        base = s * RPT

        pltpu.sync_copy(zacc_hbm, acc_sh.at[pl.ds(base, RPT)])
        pltpu.sync_copy(srcw_hbm.at[wid].at[0], src_v)
        pltpu.sync_copy(dstw_hbm.at[wid], dst_v)
        plsc.subcore_barrier()

        def gidx(j):
            return src_v.at[pl.ds(j * K, K)]

        def scat(rows, j):
            pltpu.sync_copy(rows, acc_sh.at[dst_v.at[j]], add=True)

        pltpu.async_copy(h_hbm.at[gidx(0)], rows0, sem0)

        def pair(i, _):
            j0 = 2 * i
            pltpu.async_copy(h_hbm.at[gidx(j0 + 1)], rows1, sem1)
            pltpu.make_async_copy(h_hbm.at[gidx(j0)], rows0, sem0).wait()
            scat(rows0, j0)

            @pl.when(j0 + 2 < CH)
            def _():
                pltpu.async_copy(h_hbm.at[gidx(j0 + 2)], rows0, sem0)

            pltpu.make_async_copy(h_hbm.at[gidx(j0 + 1)], rows1, sem1).wait()
            scat(rows1, j0 + 1)
            return 0

        lax.fori_loop(0, CH // 2, pair, 0)
        if CH % 2:
            pltpu.make_async_copy(h_hbm.at[gidx(CH - 1)], rows0, sem0).wait()
            scat(rows0, CH - 1)

        plsc.subcore_barrier()

        pltpu.sync_copy(acc_sh.at[pl.ds(base, RPT)],
                        p_hbm.at[c].at[pl.ds(base, RPT)])

    return pl.kernel(body, out_type=out_type, mesh=mesh,
                     scratch_types=scratch)


def _count_call():
    mesh = plsc.VectorSubcoreMesh(
        core_axis_name="c", subcore_axis_name="s", num_cores=NC, num_subcores=NS
    )
    out_type = jax.ShapeDtypeStruct((NC, NP, D), jnp.float32)
    scratch = [
        pltpu.VMEM((CH, K), jnp.int32),
        pltpu.VMEM((K, D), jnp.float32),
        pltpu.VMEM_SHARED((NP, D), jnp.float32),
    ]

    def body(dstw_hbm, ones_hbm, zacc_hbm, c_hbm, dst_v, ones_v, cnt_sh):
        c = lax.axis_index("c")
        s = lax.axis_index("s")
        wid = c * NS + s
        base = s * RPT

        pltpu.sync_copy(zacc_hbm, cnt_sh.at[pl.ds(base, RPT)])
        pltpu.sync_copy(dstw_hbm.at[wid], dst_v)
        pltpu.sync_copy(ones_hbm, ones_v)
        plsc.subcore_barrier()

        def chunk(j, _):
            pltpu.sync_copy(ones_v, cnt_sh.at[dst_v.at[j]], add=True)
            return 0

        lax.fori_loop(0, CH, chunk, 0)
        plsc.subcore_barrier()

        pltpu.sync_copy(cnt_sh.at[pl.ds(base, RPT)],
                        c_hbm.at[c].at[pl.ds(base, RPT)])

    return pl.kernel(body, out_type=out_type, mesh=mesh,
                     scratch_types=scratch)


def _mm_stats_call(h, p, cnts, wl, wr, b):
    def body(p_ref, c_ref, h_ref, wl_ref, wr_ref, b_ref, t_ref, st_ref):
        i = pl.program_id(0)
        csum = c_ref[0, :, 0:1]
        psum = p_ref[0]
        for cc in range(1, NC):
            csum = csum + c_ref[cc, :, 0:1]
            psum = psum + p_ref[cc]
        cnt = jnp.maximum(csum, 1.0)
        agg = psum / cnt
        t = (jnp.dot(agg, wl_ref[...], preferred_element_type=jnp.float32)
             + jnp.dot(h_ref[...], wr_ref[...], preferred_element_type=jnp.float32)
             + b_ref[...])
        t_ref[...] = t

        @pl.when(i == 0)
        def _():
            st_ref[...] = jnp.zeros_like(st_ref)

        st_ref[0:1, :] += jnp.sum(t, axis=0, keepdims=True)
        st_ref[1:2, :] += jnp.sum(t * t, axis=0, keepdims=True)

    return pl.pallas_call(
        body,
        grid=(NB,),
        in_specs=[
            pl.BlockSpec((NC, BR, D), lambda i: (0, i, 0)),
            pl.BlockSpec((NC, BR, D), lambda i: (0, i, 0)),
            pl.BlockSpec((BR, D), lambda i: (i, 0)),
            pl.BlockSpec((D, D), lambda i: (0, 0)),
            pl.BlockSpec((D, D), lambda i: (0, 0)),
            pl.BlockSpec((1, D), lambda i: (0, 0)),
        ],
        out_specs=[
            pl.BlockSpec((BR, D), lambda i: (i, 0)),
            pl.BlockSpec((8, D), lambda i: (0, 0)),
        ],
        out_shape=[
            jax.ShapeDtypeStruct((N, D), jnp.float32),
            jax.ShapeDtypeStruct((8, D), jnp.float32),
        ],
    )(p, cnts, h, wl, wr, b)


def _norm_call(t, st, g, bt, relu: bool):
    def body(t_ref, st_ref, g_ref, bt_ref, o_ref):
        m = st_ref[0:1, :] * (1.0 / N)
        v = st_ref[1:2, :] * (1.0 / N) - m * m
        inv = lax.rsqrt(v + EPS)
        y = (t_ref[...] - m) * (inv * g_ref[...]) + bt_ref[...]
        if relu:
            y = jnp.maximum(y, 0.0)
        o_ref[...] = y

    return pl.pallas_call(
        body,
        grid=(NB,),
        in_specs=[
            pl.BlockSpec((BR, D), lambda i: (i, 0)),
            pl.BlockSpec((8, D), lambda i: (0, 0)),
            pl.BlockSpec((1, D), lambda i: (0, 0)),
            pl.BlockSpec((1, D), lambda i: (0, 0)),
        ],
        out_specs=pl.BlockSpec((BR, D), lambda i: (i, 0)),
        out_shape=jax.ShapeDtypeStruct((N, D), jnp.float32),
    )(t, st, g, bt)


def _one(r):
    return r[0] if isinstance(r, (tuple, list)) else r


@jax.jit
def kernel(x, edge_index, W1l, W1r, b1, g1, bt1, W2l, W2r, b2, g2, bt2):
    src = edge_index[0].astype(jnp.int32).reshape(NW, 1, EW)
    dst = edge_index[1].astype(jnp.int32).reshape(NW, CH, K)
    zacc = jnp.zeros((RPT, D), jnp.float32)
    ones = jnp.ones((K, D), jnp.float32)
    b1r = b1.reshape(1, D)
    g1r = g1.reshape(1, D)
    bt1r = bt1.reshape(1, D)
    b2r = b2.reshape(1, D)
    g2r = g2.reshape(1, D)
    bt2r = bt2.reshape(1, D)

    cnts = _one(_count_call()(dst, ones, zacc))
    p1 = _one(_seg_sum_call()(x, src, dst, zacc))
    t1, st1 = _mm_stats_call(x, p1, cnts, W1l, W1r, b1r)
    h1 = _norm_call(t1, st1, g1r, bt1r, relu=True)
    p2 = _one(_seg_sum_call()(h1, src, dst, zacc))
    t2, st2 = _mm_stats_call(h1, p2, cnts, W2l, W2r, b2r)
    return _norm_call(t2, st2, g2r, bt2r, relu=False)


# speedup vs baseline: 10.0580x; 1.1259x over previous
import jax
import jax.numpy as jnp
from jax import lax
from jax.experimental import pallas as pl
from jax.experimental.pallas import tpu as pltpu
from jax.experimental.pallas import tpu_sc as plsc

N = 10000
D = 128
E = 320000
EPS = 1e-5

NC = 2
NS = 16
L = 16
NW = NC * NS
EW = E // NW
K = 80
CH = EW // K
NP = 10240
RPT = NP // NS

BR = 1000
NB = N // BR


def _seg_sum_call():
    mesh = plsc.VectorSubcoreMesh(
        core_axis_name="c", subcore_axis_name="s", num_cores=NC, num_subcores=NS
    )
    out_type = jax.ShapeDtypeStruct((NC, NP, D), jnp.float32)
    RB = 6
    NR = 3
    UN = 6
    scratch = (
        [pltpu.VMEM((RB, K), jnp.int32)]
        + [pltpu.VMEM((CH, K), jnp.int32)]
        + [pltpu.VMEM((K, D), jnp.float32) for _ in range(NR)]
        + [pltpu.VMEM_SHARED((NP, D), jnp.float32)]
        + [pltpu.SemaphoreType.DMA for _ in range(RB + 2 * NR)]
    )

    def body(h_hbm, srcw_hbm, dstw_hbm, zacc_hbm, p_hbm, *rest):
        srcc = rest[0]
        dst_v = rest[1]
        rows = rest[2:2 + NR]
        acc_sh = rest[2 + NR]
        isem = rest[3 + NR:3 + NR + RB]
        rsem = rest[3 + NR + RB:3 + NR + RB + NR]
        ssem = rest[3 + NR + RB + NR:]
        c = lax.axis_index("c")
        s = lax.axis_index("s")
        wid = c * NS + s
        base = s * RPT

        src1d = srcw_hbm.at[wid].at[0]

        def idx_copy(j, slot):
            return pltpu.make_async_copy(
                src1d.at[pl.ds(j * K, K)], srcc.at[slot], isem[slot])

        def gat_copy(j, slot, b):
            return pltpu.make_async_copy(
                h_hbm.at[srcc.at[slot]], rows[b], rsem[b])

        def sca_start(j, b):
            pltpu.async_copy(rows[b], acc_sh.at[dst_v.at[j]], ssem[b],
                             add=True)

        def sca_wait(j, b):
            pltpu.make_async_copy(rows[b], acc_sh.at[dst_v.at[j]],
                                  ssem[b]).wait()

        def guard(cond, fn):
            if isinstance(cond, bool):
                if cond:
                    fn()
            else:
                pl.when(cond)(fn)

        def step(j, t):
            b = t % NR
            bn = (t + 2) % NR
            sl = t % RB
            sln = (t + 2) % RB

            def prefetch():
                def drain_prev():
                    sca_wait(j - 1, bn)

                guard(j >= 1 if isinstance(j, int) else j >= 1, drain_prev)
                idx_copy(j + 2, sln).wait()
                gat_copy(j + 2, sln, bn).start()

            guard(j + 2 < CH, prefetch)
            gat_copy(j, sl, b).wait()
            sca_start(j, b)
            guard(j + RB < CH, lambda: idx_copy(j + RB, sl).start())

        pltpu.sync_copy(zacc_hbm, acc_sh.at[pl.ds(base, RPT)])
        pltpu.sync_copy(dstw_hbm.at[wid], dst_v)
        plsc.subcore_barrier()

        for r in range(RB):
            idx_copy(r, r).start()
        idx_copy(0, 0).wait()
        gat_copy(0, 0, 0).start()
        idx_copy(1, 1).wait()
        gat_copy(1, 1, 1).start()

        NG = CH // UN
        def group(i, _):
            j0 = i * UN
            for t in range(UN):
                step(j0 + t, t)
            return 0

        lax.fori_loop(0, NG, group, 0)
        for j in range(NG * UN, CH):
            step(j, j % UN)
        for t in range(NR):
            j = CH - 1 - t
            sca_wait(j, j % NR)

        plsc.subcore_barrier()

        pltpu.sync_copy(acc_sh.at[pl.ds(base, RPT)],
                        p_hbm.at[c].at[pl.ds(base, RPT)])

    return pl.kernel(body, out_type=out_type, mesh=mesh,
                     scratch_types=scratch)


def _count_call():
    mesh = plsc.VectorSubcoreMesh(
        core_axis_name="c", subcore_axis_name="s", num_cores=NC, num_subcores=NS
    )
    out_type = jax.ShapeDtypeStruct((NC, NP, D), jnp.float32)
    scratch = [
        pltpu.VMEM((CH, K), jnp.int32),
        pltpu.VMEM((K, D), jnp.float32),
        pltpu.VMEM_SHARED((NP, D), jnp.float32),
    ]

    def body(dstw_hbm, ones_hbm, zacc_hbm, c_hbm, dst_v, ones_v, cnt_sh):
        c = lax.axis_index("c")
        s = lax.axis_index("s")
        wid = c * NS + s
        base = s * RPT

        pltpu.sync_copy(zacc_hbm, cnt_sh.at[pl.ds(base, RPT)])
        pltpu.sync_copy(dstw_hbm.at[wid], dst_v)
        pltpu.sync_copy(ones_hbm, ones_v)
        plsc.subcore_barrier()

        def chunk(j, _):
            pltpu.sync_copy(ones_v, cnt_sh.at[dst_v.at[j]], add=True)
            return 0

        lax.fori_loop(0, CH, chunk, 0)
        plsc.subcore_barrier()

        pltpu.sync_copy(cnt_sh.at[pl.ds(base, RPT)],
                        c_hbm.at[c].at[pl.ds(base, RPT)])

    return pl.kernel(body, out_type=out_type, mesh=mesh,
                     scratch_types=scratch)


def _mm_stats_call(h, p, cnts, wl, wr, b):
    def body(p_ref, c_ref, h_ref, wl_ref, wr_ref, b_ref, t_ref, st_ref):
        i = pl.program_id(0)
        csum = c_ref[0, :, 0:1]
        psum = p_ref[0]
        for cc in range(1, NC):
            csum = csum + c_ref[cc, :, 0:1]
            psum = psum + p_ref[cc]
        cnt = jnp.maximum(csum, 1.0)
        agg = psum / cnt
        t = (jnp.dot(agg, wl_ref[...], preferred_element_type=jnp.float32)
             + jnp.dot(h_ref[...], wr_ref[...], preferred_element_type=jnp.float32)
             + b_ref[...])
        t_ref[...] = t

        @pl.when(i == 0)
        def _():
            st_ref[...] = jnp.zeros_like(st_ref)

        st_ref[0:1, :] += jnp.sum(t, axis=0, keepdims=True)
        st_ref[1:2, :] += jnp.sum(t * t, axis=0, keepdims=True)

    return pl.pallas_call(
        body,
        grid=(NB,),
        in_specs=[
            pl.BlockSpec((NC, BR, D), lambda i: (0, i, 0)),
            pl.BlockSpec((NC, BR, D), lambda i: (0, i, 0)),
            pl.BlockSpec((BR, D), lambda i: (i, 0)),
            pl.BlockSpec((D, D), lambda i: (0, 0)),
            pl.BlockSpec((D, D), lambda i: (0, 0)),
            pl.BlockSpec((1, D), lambda i: (0, 0)),
        ],
        out_specs=[
            pl.BlockSpec((BR, D), lambda i: (i, 0)),
            pl.BlockSpec((8, D), lambda i: (0, 0)),
        ],
        out_shape=[
            jax.ShapeDtypeStruct((N, D), jnp.float32),
            jax.ShapeDtypeStruct((8, D), jnp.float32),
        ],
    )(p, cnts, h, wl, wr, b)


def _norm_call(t, st, g, bt, relu: bool):
    def body(t_ref, st_ref, g_ref, bt_ref, o_ref):
        m = st_ref[0:1, :] * (1.0 / N)
        v = st_ref[1:2, :] * (1.0 / N) - m * m
        inv = lax.rsqrt(v + EPS)
        y = (t_ref[...] - m) * (inv * g_ref[...]) + bt_ref[...]
        if relu:
            y = jnp.maximum(y, 0.0)
        o_ref[...] = y

    return pl.pallas_call(
        body,
        grid=(NB,),
        in_specs=[
            pl.BlockSpec((BR, D), lambda i: (i, 0)),
            pl.BlockSpec((8, D), lambda i: (0, 0)),
            pl.BlockSpec((1, D), lambda i: (0, 0)),
            pl.BlockSpec((1, D), lambda i: (0, 0)),
        ],
        out_specs=pl.BlockSpec((BR, D), lambda i: (i, 0)),
        out_shape=jax.ShapeDtypeStruct((N, D), jnp.float32),
    )(t, st, g, bt)


def _one(r):
    return r[0] if isinstance(r, (tuple, list)) else r


@jax.jit
def kernel(x, edge_index, W1l, W1r, b1, g1, bt1, W2l, W2r, b2, g2, bt2):
    src = edge_index[0].astype(jnp.int32).reshape(NW, 1, EW)
    dst = edge_index[1].astype(jnp.int32).reshape(NW, CH, K)
    zacc = jnp.zeros((RPT, D), jnp.float32)
    ones = jnp.ones((K, D), jnp.float32)
    b1r = b1.reshape(1, D)
    g1r = g1.reshape(1, D)
    bt1r = bt1.reshape(1, D)
    b2r = b2.reshape(1, D)
    g2r = g2.reshape(1, D)
    bt2r = bt2.reshape(1, D)

    cnts = _one(_count_call()(dst, ones, zacc))
    p1 = _one(_seg_sum_call()(x, src, dst, zacc))
    t1, st1 = _mm_stats_call(x, p1, cnts, W1l, W1r, b1r)
    h1 = _norm_call(t1, st1, g1r, bt1r, relu=True)
    p2 = _one(_seg_sum_call()(h1, src, dst, zacc))
    t2, st2 = _mm_stats_call(h1, p2, cnts, W2l, W2r, b2r)
    return _norm_call(t2, st2, g2r, bt2r, relu=False)


# speedup vs baseline: 10.0979x; 1.0040x over previous
import jax
import jax.numpy as jnp
from jax import lax
from jax.experimental import pallas as pl
from jax.experimental.pallas import tpu as pltpu
from jax.experimental.pallas import tpu_sc as plsc

N = 10000
D = 128
E = 320000
EPS = 1e-5

NC = 2
NS = 16
L = 16
NW = NC * NS
EW = E // NW
K = 80
CH = EW // K
NP = 10240
RPT = NP // NS

BR = 1000
NB = N // BR


def _seg_sum_call():
    mesh = plsc.VectorSubcoreMesh(
        core_axis_name="c", subcore_axis_name="s", num_cores=NC, num_subcores=NS
    )
    out_type = jax.ShapeDtypeStruct((NC, NP, D), jnp.float32)
    RB = 6
    NR = 3
    UN = 6
    scratch = (
        [pltpu.VMEM((RB, K), jnp.int32)]
        + [pltpu.VMEM((CH, K), jnp.int32)]
        + [pltpu.VMEM((K, D), jnp.float32) for _ in range(NR)]
        + [pltpu.VMEM_SHARED((NP, D), jnp.float32)]
        + [pltpu.SemaphoreType.DMA for _ in range(RB + 2 * NR)]
    )

    def body(h_hbm, srcw_hbm, dstw_hbm, zacc_hbm, p_hbm, *rest):
        srcc = rest[0]
        dst_v = rest[1]
        rows = rest[2:2 + NR]
        acc_sh = rest[2 + NR]
        isem = rest[3 + NR:3 + NR + RB]
        rsem = rest[3 + NR + RB:3 + NR + RB + NR]
        ssem = rest[3 + NR + RB + NR:]
        c = lax.axis_index("c")
        s = lax.axis_index("s")
        wid = c * NS + s
        base = s * RPT

        src1d = srcw_hbm.at[wid].at[0]

        def idx_copy(j, slot):
            return pltpu.make_async_copy(
                src1d.at[pl.ds(j * K, K)], srcc.at[slot], isem[slot])

        def gat_copy(j, slot, b):
            return pltpu.make_async_copy(
                h_hbm.at[srcc.at[slot]], rows[b], rsem[b])

        def sca_start(j, b):
            pltpu.async_copy(rows[b], acc_sh.at[dst_v.at[j]], ssem[b],
                             add=True)

        def sca_wait(j, b):
            pltpu.make_async_copy(rows[b], acc_sh.at[dst_v.at[j]],
                                  ssem[b]).wait()

        def guard(cond, fn):
            if isinstance(cond, bool):
                if cond:
                    fn()
            else:
                pl.when(cond)(fn)

        def step(j, t):
            b = t % NR
            bn = (t + 2) % NR
            sl = t % RB
            sln = (t + 2) % RB

            def prefetch():
                def drain_prev():
                    sca_wait(j - 1, bn)

                guard(j >= 1 if isinstance(j, int) else j >= 1, drain_prev)
                idx_copy(j + 2, sln).wait()
                gat_copy(j + 2, sln, bn).start()

            guard(j + 2 < CH, prefetch)
            gat_copy(j, sl, b).wait()
            sca_start(j, b)
            guard(j + RB < CH, lambda: idx_copy(j + RB, sl).start())

        pltpu.sync_copy(zacc_hbm, acc_sh.at[pl.ds(base, RPT)])
        pltpu.sync_copy(dstw_hbm.at[wid], dst_v)
        plsc.subcore_barrier()

        for r in range(RB):
            idx_copy(r, r).start()
        idx_copy(0, 0).wait()
        gat_copy(0, 0, 0).start()
        idx_copy(1, 1).wait()
        gat_copy(1, 1, 1).start()

        NG = CH // UN
        def group(i, _):
            j0 = i * UN
            for t in range(UN):
                step(j0 + t, t)
            return 0

        lax.fori_loop(0, NG, group, 0)
        for j in range(NG * UN, CH):
            step(j, j % UN)
        for t in range(NR):
            j = CH - 1 - t
            sca_wait(j, j % NR)

        plsc.subcore_barrier()

        pltpu.sync_copy(acc_sh.at[pl.ds(base, RPT)],
                        p_hbm.at[c].at[pl.ds(base, RPT)])

    return pl.kernel(body, out_type=out_type, mesh=mesh,
                     scratch_types=scratch)


CW = 128
NSS = 4


def _count_call():
    mesh = plsc.VectorSubcoreMesh(
        core_axis_name="c", subcore_axis_name="s", num_cores=NC, num_subcores=NS
    )
    out_type = jax.ShapeDtypeStruct((NC, NP, CW), jnp.float32)
    scratch = (
        [pltpu.VMEM((CH, K), jnp.int32),
         pltpu.VMEM((K, CW), jnp.float32),
         pltpu.VMEM_SHARED((NP, CW), jnp.float32)]
        + [pltpu.SemaphoreType.DMA for _ in range(NSS)]
    )

    def body(dstw_hbm, ones_hbm, zacc_hbm, c_hbm, dst_v, ones_v, cnt_sh, *sems):
        c = lax.axis_index("c")
        s = lax.axis_index("s")
        wid = c * NS + s
        base = s * RPT

        pltpu.sync_copy(zacc_hbm, cnt_sh.at[pl.ds(base, RPT)])
        pltpu.sync_copy(dstw_hbm.at[wid], dst_v)
        pltpu.sync_copy(ones_hbm, ones_v)
        plsc.subcore_barrier()

        def sca_start(j, t):
            pltpu.async_copy(ones_v, cnt_sh.at[dst_v.at[j]], sems[t],
                             add=True)

        def sca_wait(j, t):
            pltpu.make_async_copy(ones_v, cnt_sh.at[dst_v.at[j]],
                                  sems[t]).wait()

        for j in range(NSS):
            sca_start(j, j)

        NG = (CH - NSS) // NSS

        def group(i, _):
            j0 = NSS + i * NSS
            for t in range(NSS):
                sca_wait(j0 + t - NSS, t)
                sca_start(j0 + t, t)
            return 0

        lax.fori_loop(0, NG, group, 0)
        for j in range(NSS + NG * NSS, CH):
            sca_wait(j - NSS, j % NSS)
            sca_start(j, j % NSS)
        for j in range(CH - NSS, CH):
            sca_wait(j, j % NSS)
        plsc.subcore_barrier()

        pltpu.sync_copy(cnt_sh.at[pl.ds(base, RPT)],
                        c_hbm.at[c].at[pl.ds(base, RPT)])

    return pl.kernel(body, out_type=out_type, mesh=mesh,
                     scratch_types=scratch)


def _mm_stats_call(h, p, cnts, wl, wr, b):
    def body(p_ref, c_ref, h_ref, wl_ref, wr_ref, b_ref, t_ref, st_ref):
        i = pl.program_id(0)
        csum = c_ref[0, :, 0:1]
        psum = p_ref[0]
        for cc in range(1, NC):
            csum = csum + c_ref[cc, :, 0:1]
            psum = psum + p_ref[cc]
        cnt = jnp.maximum(csum, 1.0)
        agg = psum / cnt
        t = (jnp.dot(agg, wl_ref[...], preferred_element_type=jnp.float32)
             + jnp.dot(h_ref[...], wr_ref[...], preferred_element_type=jnp.float32)
             + b_ref[...])
        t_ref[...] = t

        @pl.when(i == 0)
        def _():
            st_ref[...] = jnp.zeros_like(st_ref)

        st_ref[0:1, :] += jnp.sum(t, axis=0, keepdims=True)
        st_ref[1:2, :] += jnp.sum(t * t, axis=0, keepdims=True)

    return pl.pallas_call(
        body,
        grid=(NB,),
        in_specs=[
            pl.BlockSpec((NC, BR, D), lambda i: (0, i, 0)),
            pl.BlockSpec((NC, BR, CW), lambda i: (0, i, 0)),
            pl.BlockSpec((BR, D), lambda i: (i, 0)),
            pl.BlockSpec((D, D), lambda i: (0, 0)),
            pl.BlockSpec((D, D), lambda i: (0, 0)),
            pl.BlockSpec((1, D), lambda i: (0, 0)),
        ],
        out_specs=[
            pl.BlockSpec((BR, D), lambda i: (i, 0)),
            pl.BlockSpec((8, D), lambda i: (0, 0)),
        ],
        out_shape=[
            jax.ShapeDtypeStruct((N, D), jnp.float32),
            jax.ShapeDtypeStruct((8, D), jnp.float32),
        ],
    )(p, cnts, h, wl, wr, b)


def _norm_call(t, st, g, bt, relu: bool):
    def body(t_ref, st_ref, g_ref, bt_ref, o_ref):
        m = st_ref[0:1, :] * (1.0 / N)
        v = st_ref[1:2, :] * (1.0 / N) - m * m
        inv = lax.rsqrt(v + EPS)
        y = (t_ref[...] - m) * (inv * g_ref[...]) + bt_ref[...]
        if relu:
            y = jnp.maximum(y, 0.0)
        o_ref[...] = y

    return pl.pallas_call(
        body,
        grid=(NB,),
        in_specs=[
            pl.BlockSpec((BR, D), lambda i: (i, 0)),
            pl.BlockSpec((8, D), lambda i: (0, 0)),
            pl.BlockSpec((1, D), lambda i: (0, 0)),
            pl.BlockSpec((1, D), lambda i: (0, 0)),
        ],
        out_specs=pl.BlockSpec((BR, D), lambda i: (i, 0)),
        out_shape=jax.ShapeDtypeStruct((N, D), jnp.float32),
    )(t, st, g, bt)


def _one(r):
    return r[0] if isinstance(r, (tuple, list)) else r


@jax.jit
def kernel(x, edge_index, W1l, W1r, b1, g1, bt1, W2l, W2r, b2, g2, bt2):
    src = edge_index[0].astype(jnp.int32).reshape(NW, 1, EW)
    dst = edge_index[1].astype(jnp.int32).reshape(NW, CH, K)
    zacc = jnp.zeros((RPT, D), jnp.float32)
    zacc_c = jnp.zeros((RPT, CW), jnp.float32)
    ones = jnp.ones((K, CW), jnp.float32)
    b1r = b1.reshape(1, D)
    g1r = g1.reshape(1, D)
    bt1r = bt1.reshape(1, D)
    b2r = b2.reshape(1, D)
    g2r = g2.reshape(1, D)
    bt2r = bt2.reshape(1, D)

    cnts = _one(_count_call()(dst, ones, zacc_c))
    p1 = _one(_seg_sum_call()(x, src, dst, zacc))
    t1, st1 = _mm_stats_call(x, p1, cnts, W1l, W1r, b1r)
    h1 = _norm_call(t1, st1, g1r, bt1r, relu=True)
    p2 = _one(_seg_sum_call()(h1, src, dst, zacc))
    t2, st2 = _mm_stats_call(h1, p2, cnts, W2l, W2r, b2r)
    return _norm_call(t2, st2, g2r, bt2r, relu=False)


# speedup vs baseline: 10.1091x; 1.0011x over previous
import jax
import jax.numpy as jnp
from jax import lax
from jax.experimental import pallas as pl
from jax.experimental.pallas import tpu as pltpu
from jax.experimental.pallas import tpu_sc as plsc

N = 10000
D = 128
E = 320000
EPS = 1e-5

NC = 2
NS = 16
L = 16
NW = NC * NS
EW = E // NW
K = 80
CH = EW // K
NP = 10240
RPT = NP // NS

BR = 1000
NB = N // BR


def _seg_sum_call():
    mesh = plsc.VectorSubcoreMesh(
        core_axis_name="c", subcore_axis_name="s", num_cores=NC, num_subcores=NS
    )
    out_type = jax.ShapeDtypeStruct((NC, NP, D), jnp.float32)
    RB = 6
    NR = 3
    UN = 6
    scratch = (
        [pltpu.VMEM((RB, K), jnp.int32)]
        + [pltpu.VMEM((CH, K), jnp.int32)]
        + [pltpu.VMEM((K, D), jnp.float32) for _ in range(NR)]
        + [pltpu.VMEM_SHARED((NP, D), jnp.float32)]
        + [pltpu.SemaphoreType.DMA for _ in range(RB + 2 * NR)]
    )

    def body(h_hbm, srcw_hbm, dstw_hbm, zacc_hbm, p_hbm, *rest):
        srcc = rest[0]
        dst_v = rest[1]
        rows = rest[2:2 + NR]
        acc_sh = rest[2 + NR]
        isem = rest[3 + NR:3 + NR + RB]
        rsem = rest[3 + NR + RB:3 + NR + RB + NR]
        ssem = rest[3 + NR + RB + NR:]
        c = lax.axis_index("c")
        s = lax.axis_index("s")
        wid = c * NS + s
        base = s * RPT

        src1d = srcw_hbm.at[wid].at[0]

        def idx_copy(j, slot):
            return pltpu.make_async_copy(
                src1d.at[pl.ds(j * K, K)], srcc.at[slot], isem[slot])

        def gat_copy(j, slot, b):
            return pltpu.make_async_copy(
                h_hbm.at[srcc.at[slot]], rows[b], rsem[b])

        def sca_start(j, b):
            pltpu.async_copy(rows[b], acc_sh.at[dst_v.at[j]], ssem[b],
                             add=True)

        def sca_wait(j, b):
            pltpu.make_async_copy(rows[b], acc_sh.at[dst_v.at[j]],
                                  ssem[b]).wait()

        def guard(cond, fn):
            if isinstance(cond, bool):
                if cond:
                    fn()
            else:
                pl.when(cond)(fn)

        def step(j, t):
            b = t % NR
            bn = (t + 2) % NR
            sl = t % RB
            sln = (t + 2) % RB

            def prefetch():
                def drain_prev():
                    sca_wait(j - 1, bn)

                guard(j >= 1 if isinstance(j, int) else j >= 1, drain_prev)
                idx_copy(j + 2, sln).wait()
                gat_copy(j + 2, sln, bn).start()

            guard(j + 2 < CH, prefetch)
            gat_copy(j, sl, b).wait()
            sca_start(j, b)
            guard(j + RB < CH, lambda: idx_copy(j + RB, sl).start())

        pltpu.sync_copy(zacc_hbm, acc_sh.at[pl.ds(base, RPT)])
        pltpu.sync_copy(dstw_hbm.at[wid], dst_v)
        plsc.subcore_barrier()

        for r in range(RB):
            idx_copy(r, r).start()
        idx_copy(0, 0).wait()
        gat_copy(0, 0, 0).start()
        idx_copy(1, 1).wait()
        gat_copy(1, 1, 1).start()

        NG = CH // UN
        def group(i, _):
            j0 = i * UN
            for t in range(UN):
                step(j0 + t, t)
            return 0

        lax.fori_loop(0, NG, group, 0)
        for j in range(NG * UN, CH):
            step(j, j % UN)
        for t in range(NR):
            j = CH - 1 - t
            sca_wait(j, j % NR)

        plsc.subcore_barrier()

        pltpu.sync_copy(acc_sh.at[pl.ds(base, RPT)],
                        p_hbm.at[c].at[pl.ds(base, RPT)])

    return pl.kernel(body, out_type=out_type, mesh=mesh,
                     scratch_types=scratch)


CW = 128
NSS = 4


def _count_call():
    mesh = plsc.VectorSubcoreMesh(
        core_axis_name="c", subcore_axis_name="s", num_cores=NC, num_subcores=NS
    )
    out_type = jax.ShapeDtypeStruct((NC, NP, CW), jnp.float32)
    scratch = (
        [pltpu.VMEM((CH, K), jnp.int32),
         pltpu.VMEM((K, CW), jnp.float32),
         pltpu.VMEM_SHARED((NP, CW), jnp.float32)]
        + [pltpu.SemaphoreType.DMA for _ in range(NSS)]
    )

    def body(dstw_hbm, ones_hbm, zacc_hbm, c_hbm, dst_v, ones_v, cnt_sh, *sems):
        c = lax.axis_index("c")
        s = lax.axis_index("s")
        wid = c * NS + s
        base = s * RPT

        pltpu.sync_copy(zacc_hbm, cnt_sh.at[pl.ds(base, RPT)])
        pltpu.sync_copy(dstw_hbm.at[wid], dst_v)
        pltpu.sync_copy(ones_hbm, ones_v)
        plsc.subcore_barrier()

        def sca_start(j, t):
            pltpu.async_copy(ones_v, cnt_sh.at[dst_v.at[j]], sems[t],
                             add=True)

        def sca_wait(j, t):
            pltpu.make_async_copy(ones_v, cnt_sh.at[dst_v.at[j]],
                                  sems[t]).wait()

        for j in range(NSS):
            sca_start(j, j)

        NG = (CH - NSS) // NSS

        def group(i, _):
            j0 = NSS + i * NSS
            for t in range(NSS):
                sca_wait(j0 + t - NSS, t)
                sca_start(j0 + t, t)
            return 0

        lax.fori_loop(0, NG, group, 0)
        for j in range(NSS + NG * NSS, CH):
            sca_wait(j - NSS, j % NSS)
            sca_start(j, j % NSS)
        for j in range(CH - NSS, CH):
            sca_wait(j, j % NSS)
        plsc.subcore_barrier()

        pltpu.sync_copy(cnt_sh.at[pl.ds(base, RPT)],
                        c_hbm.at[c].at[pl.ds(base, RPT)])

    return pl.kernel(body, out_type=out_type, mesh=mesh,
                     scratch_types=scratch)


def _mm_stats_call(h, p, cnts, wl, wr, b, cnt_narrow: bool):
    def body(p_ref, c_ref, h_ref, wl_ref, wr_ref, b_ref, t_ref, st_ref,
             *cn_ref):
        i = pl.program_id(0)
        psum = p_ref[0]
        for cc in range(1, NC):
            psum = psum + p_ref[cc]
        if cnt_narrow:
            cnt = c_ref[...]
        else:
            csum = c_ref[0, :, 0:1]
            for cc in range(1, NC):
                csum = csum + c_ref[cc, :, 0:1]
            cnt = jnp.maximum(csum, 1.0)
            cn_ref[0][...] = cnt
        agg = psum / cnt
        t = (jnp.dot(agg, wl_ref[...], preferred_element_type=jnp.float32)
             + jnp.dot(h_ref[...], wr_ref[...], preferred_element_type=jnp.float32)
             + b_ref[...])
        t_ref[...] = t

        @pl.when(i == 0)
        def _():
            st_ref[...] = jnp.zeros_like(st_ref)

        st_ref[0:1, :] += jnp.sum(t, axis=0, keepdims=True)
        st_ref[1:2, :] += jnp.sum(t * t, axis=0, keepdims=True)

    if cnt_narrow:
        c_spec = pl.BlockSpec((BR, 1), lambda i: (i, 0))
        extra_out_specs = []
        extra_out_shapes = []
    else:
        c_spec = pl.BlockSpec((NC, BR, CW), lambda i: (0, i, 0))
        extra_out_specs = [pl.BlockSpec((BR, 1), lambda i: (i, 0))]
        extra_out_shapes = [jax.ShapeDtypeStruct((N, 1), jnp.float32)]

    return pl.pallas_call(
        body,
        grid=(NB,),
        in_specs=[
            pl.BlockSpec((NC, BR, D), lambda i: (0, i, 0)),
            c_spec,
            pl.BlockSpec((BR, D), lambda i: (i, 0)),
            pl.BlockSpec((D, D), lambda i: (0, 0)),
            pl.BlockSpec((D, D), lambda i: (0, 0)),
            pl.BlockSpec((1, D), lambda i: (0, 0)),
        ],
        out_specs=[
            pl.BlockSpec((BR, D), lambda i: (i, 0)),
            pl.BlockSpec((8, D), lambda i: (0, 0)),
        ] + extra_out_specs,
        out_shape=[
            jax.ShapeDtypeStruct((N, D), jnp.float32),
            jax.ShapeDtypeStruct((8, D), jnp.float32),
        ] + extra_out_shapes,
    )(p, cnts, h, wl, wr, b)


def _norm_call(t, st, g, bt, relu: bool):
    def body(t_ref, st_ref, g_ref, bt_ref, o_ref):
        m = st_ref[0:1, :] * (1.0 / N)
        v = st_ref[1:2, :] * (1.0 / N) - m * m
        inv = lax.rsqrt(v + EPS)
        y = (t_ref[...] - m) * (inv * g_ref[...]) + bt_ref[...]
        if relu:
            y = jnp.maximum(y, 0.0)
        o_ref[...] = y

    return pl.pallas_call(
        body,
        grid=(NB,),
        in_specs=[
            pl.BlockSpec((BR, D), lambda i: (i, 0)),
            pl.BlockSpec((8, D), lambda i: (0, 0)),
            pl.BlockSpec((1, D), lambda i: (0, 0)),
            pl.BlockSpec((1, D), lambda i: (0, 0)),
        ],
        out_specs=pl.BlockSpec((BR, D), lambda i: (i, 0)),
        out_shape=jax.ShapeDtypeStruct((N, D), jnp.float32),
    )(t, st, g, bt)


def _one(r):
    return r[0] if isinstance(r, (tuple, list)) else r


@jax.jit
def kernel(x, edge_index, W1l, W1r, b1, g1, bt1, W2l, W2r, b2, g2, bt2):
    src = edge_index[0].astype(jnp.int32).reshape(NW, 1, EW)
    dst = edge_index[1].astype(jnp.int32).reshape(NW, CH, K)
    zacc = jnp.zeros((RPT, D), jnp.float32)
    zacc_c = jnp.zeros((RPT, CW), jnp.float32)
    ones = jnp.ones((K, CW), jnp.float32)
    b1r = b1.reshape(1, D)
    g1r = g1.reshape(1, D)
    bt1r = bt1.reshape(1, D)
    b2r = b2.reshape(1, D)
    g2r = g2.reshape(1, D)
    bt2r = bt2.reshape(1, D)

    cnts = _one(_count_call()(dst, ones, zacc_c))
    p1 = _one(_seg_sum_call()(x, src, dst, zacc))
    t1, st1, cnt1 = _mm_stats_call(x, p1, cnts, W1l, W1r, b1r,
                                   cnt_narrow=False)
    h1 = _norm_call(t1, st1, g1r, bt1r, relu=True)
    p2 = _one(_seg_sum_call()(h1, src, dst, zacc))
    t2, st2 = _mm_stats_call(h1, p2, cnt1, W2l, W2r, b2r, cnt_narrow=True)
    return _norm_call(t2, st2, g2r, bt2r, relu=False)
